# Initial kernel scaffold; baseline (speedup 1.0000x reference)
import jax
import jax.numpy as jnp
from jax.experimental import pallas as pl


def kernel(vecs, loss_mask):
    raise NotImplementedError("write your pallas kernel here")


# speedup vs baseline: 1.7279x; 1.7279x over previous
import functools

import jax
import jax.numpy as jnp
from jax.experimental import pallas as pl

D_K = 64
N_CODE = 1024
PE_LAM = 100000.0
TAU = float(D_K) ** 0.5

T_DIM, B_DIM, H_DIM, L_DIM = 8, 8, 1, 512
N_ROWS = T_DIM * B_DIM * H_DIM * L_DIM
ROW_TILE = 512
N_TILES = N_ROWS // ROW_TILE


def _codebook():
    pos = jnp.arange(N_CODE, dtype=jnp.float32)
    inv_lams = 1.0 / (PE_LAM ** (jnp.arange(0, D_K, 2, dtype=jnp.float32) / D_K))
    pre = pos[:, None] * inv_lams[None, :]
    cat = jnp.concatenate([jnp.sin(pre), jnp.cos(pre)], axis=-1)
    rms = cat * jax.lax.rsqrt(jnp.mean(jnp.square(cat), axis=-1, keepdims=True) + 1e-6)
    return (TAU ** -0.5) * jax.lax.stop_gradient(rms)[None, ...]


def _vq_tile_kernel(v_ref, vn_ref, ct_ref, c_ref, cn_ref, mask_ref,
                    hat_ref, z_ref, err_ref, acc_ref):
    i = pl.program_id(0)
    v = v_ref[...]
    dot = jnp.dot(v, ct_ref[...], preferred_element_type=jnp.float32)
    diffs2 = (vn_ref[...] - 2.0 * dot) + cn_ref[...]
    err = jnp.min(diffs2, axis=-1)
    iota = jax.lax.broadcasted_iota(jnp.int32, diffs2.shape, 1)
    z = jnp.min(
        jnp.where(diffs2 == err[:, None], iota, jnp.int32(N_CODE)), axis=-1
    ).astype(jnp.int32)
    onehot = (iota == z[:, None]).astype(jnp.float32)
    hat_ref[...] = jnp.dot(onehot, c_ref[...],
                           preferred_element_type=jnp.float32,
                           precision=jax.lax.Precision.HIGHEST)
    z_ref[...] = z[None, None, :]
    err_ref[...] = err[None, None, :]
    part = jnp.sum(mask_ref[...] * err[None, None, :]).reshape(1, 1)

    @pl.when(i == 0)
    def _():
        acc_ref[...] = jnp.zeros_like(acc_ref)

    acc_ref[...] += part


@jax.jit
def kernel(vecs, loss_mask):
    orig_dtype = vecs.dtype
    vecs_hp = vecs.astype(jnp.float32)
    v = vecs_hp.reshape(N_ROWS, D_K)
    c3 = _codebook()
    c = c3[0]
    ct = c.T
    cnorm = jnp.sum(jnp.square(c3), axis=-1)[0][None, :]
    vnorm = jnp.sum(jnp.square(vecs_hp), axis=-1).reshape(N_ROWS, 1)
    mask_rows = jnp.broadcast_to(
        loss_mask[:, None, :, :], (T_DIM, B_DIM, H_DIM, L_DIM)
    ).reshape(N_TILES, 1, ROW_TILE)

    grid = (N_TILES,)
    hat, z, err, acc = pl.pallas_call(
        _vq_tile_kernel,
        grid=grid,
        in_specs=[
            pl.BlockSpec((ROW_TILE, D_K), lambda i: (i, 0)),
            pl.BlockSpec((ROW_TILE, 1), lambda i: (i, 0)),
            pl.BlockSpec((D_K, N_CODE), lambda i: (0, 0)),
            pl.BlockSpec((N_CODE, D_K), lambda i: (0, 0)),
            pl.BlockSpec((1, N_CODE), lambda i: (0, 0)),
            pl.BlockSpec((1, 1, ROW_TILE), lambda i: (i, 0, 0)),
        ],
        out_specs=[
            pl.BlockSpec((ROW_TILE, D_K), lambda i: (i, 0)),
            pl.BlockSpec((1, 1, ROW_TILE), lambda i: (i, 0, 0)),
            pl.BlockSpec((1, 1, ROW_TILE), lambda i: (i, 0, 0)),
            pl.BlockSpec((1, 1), lambda i: (0, 0)),
        ],
        out_shape=[
            jax.ShapeDtypeStruct((N_ROWS, D_K), jnp.float32),
            jax.ShapeDtypeStruct((N_TILES, 1, ROW_TILE), jnp.int32),
            jax.ShapeDtypeStruct((N_TILES, 1, ROW_TILE), jnp.float32),
            jax.ShapeDtypeStruct((1, 1), jnp.float32),
        ],
    )(v, vnorm, ct, c, cnorm, mask_rows)

    vecs_hat = hat.reshape(T_DIM, B_DIM, H_DIM, L_DIM, D_K).astype(orig_dtype)
    z_out = z.reshape(T_DIM, B_DIM, H_DIM, L_DIM)
    errs2 = err.reshape(T_DIM, B_DIM, H_DIM, L_DIM)
    l_commit = acc[0, 0] / jnp.float32(T_DIM * H_DIM * L_DIM)
    l_codebook = jnp.zeros((), dtype=jnp.float32)
    return vecs_hat, z_out, l_commit, l_codebook, errs2


# speedup vs baseline: 2.2266x; 1.2886x over previous
import functools

import jax
import jax.numpy as jnp
from jax.experimental import pallas as pl

D_K = 64
N_CODE = 1024
PE_LAM = 100000.0
TAU = float(D_K) ** 0.5

T_DIM, B_DIM, H_DIM, L_DIM = 8, 8, 1, 512
N_ROWS = T_DIM * B_DIM * H_DIM * L_DIM
ROW_TILE = 512
N_TILES = N_ROWS // ROW_TILE


def _codebook():
    pos = jnp.arange(N_CODE, dtype=jnp.float32)
    inv_lams = 1.0 / (PE_LAM ** (jnp.arange(0, D_K, 2, dtype=jnp.float32) / D_K))
    pre = pos[:, None] * inv_lams[None, :]
    cat = jnp.concatenate([jnp.sin(pre), jnp.cos(pre)], axis=-1)
    rms = cat * jax.lax.rsqrt(jnp.mean(jnp.square(cat), axis=-1, keepdims=True) + 1e-6)
    return (TAU ** -0.5) * jax.lax.stop_gradient(rms)[None, ...]


def _vq_tile_kernel(v_ref, vn_ref, ct_ref, chi_ref, cmid_ref, clo_ref, cn_ref,
                    mask_ref, hat_ref, z_ref, err_ref, acc_ref):
    i = pl.program_id(0)
    v = v_ref[...]
    dot = jnp.dot(v, ct_ref[...], preferred_element_type=jnp.float32)
    diffs2 = (vn_ref[...] - 2.0 * dot) + cn_ref[...]
    err = jnp.min(diffs2, axis=-1)
    iota = jax.lax.broadcasted_iota(jnp.int32, diffs2.shape, 1)
    z = jnp.min(
        jnp.where(diffs2 == err[:, None], iota, jnp.int32(N_CODE)), axis=-1
    ).astype(jnp.int32)
    onehot = (iota == z[:, None]).astype(jnp.bfloat16)
    hat = jnp.dot(onehot, chi_ref[...], preferred_element_type=jnp.float32)
    hat += jnp.dot(onehot, cmid_ref[...], preferred_element_type=jnp.float32)
    hat += jnp.dot(onehot, clo_ref[...], preferred_element_type=jnp.float32)
    hat_ref[...] = hat
    z_ref[...] = z[None, None, :]
    err_ref[...] = err[None, None, :]
    part = jnp.sum(mask_ref[...] * err[None, None, :]).reshape(1, 1)

    @pl.when(i == 0)
    def _():
        acc_ref[...] = jnp.zeros_like(acc_ref)

    acc_ref[...] += part


@jax.jit
def kernel(vecs, loss_mask):
    orig_dtype = vecs.dtype
    vecs_hp = vecs.astype(jnp.float32)
    v = vecs_hp.reshape(N_ROWS, D_K)
    c3 = _codebook()
    c = c3[0]
    ct = c.T
    cnorm = jnp.sum(jnp.square(c3), axis=-1)[0][None, :]
    c_hi = c.astype(jnp.bfloat16)
    r1 = c - c_hi.astype(jnp.float32)
    c_mid = r1.astype(jnp.bfloat16)
    c_lo = (r1 - c_mid.astype(jnp.float32)).astype(jnp.bfloat16)
    vnorm = jnp.sum(jnp.square(vecs_hp), axis=-1).reshape(N_ROWS, 1)
    mask_rows = jnp.broadcast_to(
        loss_mask[:, None, :, :], (T_DIM, B_DIM, H_DIM, L_DIM)
    ).reshape(N_TILES, 1, ROW_TILE)

    grid = (N_TILES,)
    hat, z, err, acc = pl.pallas_call(
        _vq_tile_kernel,
        grid=grid,
        in_specs=[
            pl.BlockSpec((ROW_TILE, D_K), lambda i: (i, 0)),
            pl.BlockSpec((ROW_TILE, 1), lambda i: (i, 0)),
            pl.BlockSpec((D_K, N_CODE), lambda i: (0, 0)),
            pl.BlockSpec((N_CODE, D_K), lambda i: (0, 0)),
            pl.BlockSpec((N_CODE, D_K), lambda i: (0, 0)),
            pl.BlockSpec((N_CODE, D_K), lambda i: (0, 0)),
            pl.BlockSpec((1, N_CODE), lambda i: (0, 0)),
            pl.BlockSpec((1, 1, ROW_TILE), lambda i: (i, 0, 0)),
        ],
        out_specs=[
            pl.BlockSpec((ROW_TILE, D_K), lambda i: (i, 0)),
            pl.BlockSpec((1, 1, ROW_TILE), lambda i: (i, 0, 0)),
            pl.BlockSpec((1, 1, ROW_TILE), lambda i: (i, 0, 0)),
            pl.BlockSpec((1, 1), lambda i: (0, 0)),
        ],
        out_shape=[
            jax.ShapeDtypeStruct((N_ROWS, D_K), jnp.float32),
            jax.ShapeDtypeStruct((N_TILES, 1, ROW_TILE), jnp.int32),
            jax.ShapeDtypeStruct((N_TILES, 1, ROW_TILE), jnp.float32),
            jax.ShapeDtypeStruct((1, 1), jnp.float32),
        ],
    )(v, vnorm, ct, c_hi, c_mid, c_lo, cnorm, mask_rows)

    vecs_hat = hat.reshape(T_DIM, B_DIM, H_DIM, L_DIM, D_K).astype(orig_dtype)
    z_out = z.reshape(T_DIM, B_DIM, H_DIM, L_DIM)
    errs2 = err.reshape(T_DIM, B_DIM, H_DIM, L_DIM)
    l_commit = acc[0, 0] / jnp.float32(T_DIM * H_DIM * L_DIM)
    l_codebook = jnp.zeros((), dtype=jnp.float32)
    return vecs_hat, z_out, l_commit, l_codebook, errs2


# speedup vs baseline: 2.2318x; 1.0024x over previous
import functools

import jax
import jax.numpy as jnp
from jax.experimental import pallas as pl

D_K = 64
N_CODE = 1024
PE_LAM = 100000.0
TAU = float(D_K) ** 0.5

T_DIM, B_DIM, H_DIM, L_DIM = 8, 8, 1, 512
N_ROWS = T_DIM * B_DIM * H_DIM * L_DIM
ROW_TILE = 512
N_TILES = N_ROWS // ROW_TILE


def _codebook():
    pos = jnp.arange(N_CODE, dtype=jnp.float32)
    inv_lams = 1.0 / (PE_LAM ** (jnp.arange(0, D_K, 2, dtype=jnp.float32) / D_K))
    pre = pos[:, None] * inv_lams[None, :]
    cat = jnp.concatenate([jnp.sin(pre), jnp.cos(pre)], axis=-1)
    rms = cat * jax.lax.rsqrt(jnp.mean(jnp.square(cat), axis=-1, keepdims=True) + 1e-6)
    return (TAU ** -0.5) * jax.lax.stop_gradient(rms)[None, ...]


def _vq_tile_kernel(v_ref, vn_ref, ct_ref, chi_ref, cmid_ref, clo_ref, cn_ref,
                    mask_ref, hat_ref, z_ref, err_ref, acc_ref):
    i = pl.program_id(0)
    v = v_ref[...]
    dot = jnp.dot(v, ct_ref[...], preferred_element_type=jnp.float32)
    diffs2 = (vn_ref[...] - 2.0 * dot) + cn_ref[...]
    err = jnp.min(diffs2, axis=-1)
    iota = jax.lax.broadcasted_iota(jnp.int32, diffs2.shape, 1)
    z = jnp.min(
        jnp.where(diffs2 == err[:, None], iota, jnp.int32(N_CODE)), axis=-1
    ).astype(jnp.int32)
    onehot = (iota == z[:, None]).astype(jnp.bfloat16)
    hat = jnp.dot(onehot, chi_ref[...], preferred_element_type=jnp.float32)
    hat += jnp.dot(onehot, cmid_ref[...], preferred_element_type=jnp.float32)
    hat += jnp.dot(onehot, clo_ref[...], preferred_element_type=jnp.float32)
    hat_ref[...] = hat
    z_ref[...] = z[None, None, :]
    err_ref[...] = err[None, None, :]
    part = jnp.sum(mask_ref[...] * err[None, None, :]).reshape(1, 1)

    @pl.when(i == 0)
    def _():
        acc_ref[...] = jnp.zeros_like(acc_ref)

    acc_ref[...] += part


@jax.jit
def kernel(vecs, loss_mask):
    orig_dtype = vecs.dtype
    vecs_hp = vecs.astype(jnp.float32)
    v = vecs_hp.reshape(N_ROWS, D_K)
    c3 = _codebook()
    c = c3[0]
    ct = c.T
    cnorm = jnp.sum(jnp.square(c3), axis=-1)[0][None, :]
    c_hi = c.astype(jnp.bfloat16)
    r1 = c - jax.lax.optimization_barrier(c_hi).astype(jnp.float32)
    c_mid = r1.astype(jnp.bfloat16)
    c_lo = (r1 - jax.lax.optimization_barrier(c_mid).astype(jnp.float32)
            ).astype(jnp.bfloat16)
    vnorm = jnp.sum(jnp.square(vecs_hp), axis=-1).reshape(N_ROWS, 1)
    mask_rows = jnp.broadcast_to(
        loss_mask[:, None, :, :], (T_DIM, B_DIM, H_DIM, L_DIM)
    ).reshape(N_TILES, 1, ROW_TILE)

    grid = (N_TILES,)
    hat, z, err, acc = pl.pallas_call(
        _vq_tile_kernel,
        grid=grid,
        in_specs=[
            pl.BlockSpec((ROW_TILE, D_K), lambda i: (i, 0)),
            pl.BlockSpec((ROW_TILE, 1), lambda i: (i, 0)),
            pl.BlockSpec((D_K, N_CODE), lambda i: (0, 0)),
            pl.BlockSpec((N_CODE, D_K), lambda i: (0, 0)),
            pl.BlockSpec((N_CODE, D_K), lambda i: (0, 0)),
            pl.BlockSpec((N_CODE, D_K), lambda i: (0, 0)),
            pl.BlockSpec((1, N_CODE), lambda i: (0, 0)),
            pl.BlockSpec((1, 1, ROW_TILE), lambda i: (i, 0, 0)),
        ],
        out_specs=[
            pl.BlockSpec((ROW_TILE, D_K), lambda i: (i, 0)),
            pl.BlockSpec((1, 1, ROW_TILE), lambda i: (i, 0, 0)),
            pl.BlockSpec((1, 1, ROW_TILE), lambda i: (i, 0, 0)),
            pl.BlockSpec((1, 1), lambda i: (0, 0)),
        ],
        out_shape=[
            jax.ShapeDtypeStruct((N_ROWS, D_K), jnp.float32),
            jax.ShapeDtypeStruct((N_TILES, 1, ROW_TILE), jnp.int32),
            jax.ShapeDtypeStruct((N_TILES, 1, ROW_TILE), jnp.float32),
            jax.ShapeDtypeStruct((1, 1), jnp.float32),
        ],
    )(v, vnorm, ct, c_hi, c_mid, c_lo, cnorm, mask_rows)

    vecs_hat = hat.reshape(T_DIM, B_DIM, H_DIM, L_DIM, D_K).astype(orig_dtype)
    z_out = z.reshape(T_DIM, B_DIM, H_DIM, L_DIM)
    errs2 = err.reshape(T_DIM, B_DIM, H_DIM, L_DIM)
    l_commit = acc[0, 0] / jnp.float32(T_DIM * H_DIM * L_DIM)
    l_codebook = jnp.zeros((), dtype=jnp.float32)
    return vecs_hat, z_out, l_commit, l_codebook, errs2


# speedup vs baseline: 2.3300x; 1.0440x over previous
import jax
import jax.numpy as jnp
from jax.experimental import pallas as pl

D_K = 64
N_CODE = 1024
PE_LAM = 100000.0
TAU = float(D_K) ** 0.5

T_DIM, B_DIM, H_DIM, L_DIM = 8, 8, 1, 512


def _codebook():
    pos = jnp.arange(N_CODE, dtype=jnp.float32)
    inv_lams = 1.0 / (PE_LAM ** (jnp.arange(0, D_K, 2, dtype=jnp.float32) / D_K))
    pre = pos[:, None] * inv_lams[None, :]
    cat = jnp.concatenate([jnp.sin(pre), jnp.cos(pre)], axis=-1)
    rms = cat * jax.lax.rsqrt(jnp.mean(jnp.square(cat), axis=-1, keepdims=True) + 1e-6)
    return (TAU ** -0.5) * jax.lax.stop_gradient(rms)[None, ...]


def _vq_tile_kernel(v_ref, vn_ref, ct_ref, chi_ref, cmid_ref, clo_ref, cn_ref,
                    hat_ref, z_ref, err_ref):
    v = v_ref[0, 0, 0]
    dot = jnp.dot(v, ct_ref[...], preferred_element_type=jnp.float32)
    diffs2 = (vn_ref[0, 0, 0] - 2.0 * dot) + cn_ref[...]
    err = jnp.min(diffs2, axis=-1)
    iota = jax.lax.broadcasted_iota(jnp.int32, diffs2.shape, 1)
    z = jnp.min(
        jnp.where(diffs2 == err[:, None], iota, jnp.int32(N_CODE)), axis=-1
    ).astype(jnp.int32)
    onehot = (iota == z[:, None]).astype(jnp.bfloat16)
    hat = jnp.dot(onehot, chi_ref[...], preferred_element_type=jnp.float32)
    hat += jnp.dot(onehot, cmid_ref[...], preferred_element_type=jnp.float32)
    hat += jnp.dot(onehot, clo_ref[...], preferred_element_type=jnp.float32)
    hat_ref[0, 0, 0] = hat
    z_ref[0, 0, 0] = z
    err_ref[0, 0, 0] = err


@jax.jit
def kernel(vecs, loss_mask):
    orig_dtype = vecs.dtype
    vecs_hp = vecs.astype(jnp.float32)
    c3 = _codebook()
    c = c3[0]
    ct = c.T
    cnorm = jnp.sum(jnp.square(c3), axis=-1)[0][None, :]
    c_hi = c.astype(jnp.bfloat16)
    r1 = c - jax.lax.optimization_barrier(c_hi).astype(jnp.float32)
    c_mid = r1.astype(jnp.bfloat16)
    c_lo = (r1 - jax.lax.optimization_barrier(c_mid).astype(jnp.float32)
            ).astype(jnp.bfloat16)
    vnorm = jnp.sum(jnp.square(vecs_hp), axis=-1)[..., None]

    grid = (T_DIM, B_DIM)
    hat, z, err = pl.pallas_call(
        _vq_tile_kernel,
        grid=grid,
        in_specs=[
            pl.BlockSpec((1, 1, H_DIM, L_DIM, D_K), lambda t, b: (t, b, 0, 0, 0)),
            pl.BlockSpec((1, 1, H_DIM, L_DIM, 1), lambda t, b: (t, b, 0, 0, 0)),
            pl.BlockSpec((D_K, N_CODE), lambda t, b: (0, 0)),
            pl.BlockSpec((N_CODE, D_K), lambda t, b: (0, 0)),
            pl.BlockSpec((N_CODE, D_K), lambda t, b: (0, 0)),
            pl.BlockSpec((N_CODE, D_K), lambda t, b: (0, 0)),
            pl.BlockSpec((1, N_CODE), lambda t, b: (0, 0)),
        ],
        out_specs=[
            pl.BlockSpec((1, 1, H_DIM, L_DIM, D_K), lambda t, b: (t, b, 0, 0, 0)),
            pl.BlockSpec((1, 1, H_DIM, L_DIM), lambda t, b: (t, b, 0, 0)),
            pl.BlockSpec((1, 1, H_DIM, L_DIM), lambda t, b: (t, b, 0, 0)),
        ],
        out_shape=[
            jax.ShapeDtypeStruct((T_DIM, B_DIM, H_DIM, L_DIM, D_K), jnp.float32),
            jax.ShapeDtypeStruct((T_DIM, B_DIM, H_DIM, L_DIM), jnp.int32),
            jax.ShapeDtypeStruct((T_DIM, B_DIM, H_DIM, L_DIM), jnp.float32),
        ],
    )(vecs_hp, vnorm, ct, c_hi, c_mid, c_lo, cnorm)

    vecs_hat = hat.astype(orig_dtype)
    errs2 = err
    l_commit = jnp.mean(jnp.sum(loss_mask[:, None, :, :] * errs2, axis=1))
    l_codebook = jnp.zeros((), dtype=jnp.float32)
    return vecs_hat, z, l_commit, l_codebook, errs2


# speedup vs baseline: 2.6827x; 1.1514x over previous
import jax
import jax.numpy as jnp
from jax.experimental import pallas as pl

D_K = 64
N_CODE = 1024
PE_LAM = 100000.0
TAU = float(D_K) ** 0.5

T_DIM, B_DIM, H_DIM, L_DIM = 8, 8, 1, 512


def _codebook():
    pos = jnp.arange(N_CODE, dtype=jnp.float32)
    inv_lams = 1.0 / (PE_LAM ** (jnp.arange(0, D_K, 2, dtype=jnp.float32) / D_K))
    pre = pos[:, None] * inv_lams[None, :]
    cat = jnp.concatenate([jnp.sin(pre), jnp.cos(pre)], axis=-1)
    rms = cat * jax.lax.rsqrt(jnp.mean(jnp.square(cat), axis=-1, keepdims=True) + 1e-6)
    return (TAU ** -0.5) * jax.lax.stop_gradient(rms)[None, ...]


def _vq_tile_kernel(v_ref, vn_ref, ct_ref, chiz_ref, cmid_ref, clo_ref, cn_ref,
                    hat_ref, z_ref, err_ref):
    v = v_ref[0, 0, 0]
    dot = jnp.dot(v, ct_ref[...], preferred_element_type=jnp.float32)
    diffs2 = (vn_ref[0, 0, 0] - 2.0 * dot) + cn_ref[...]
    err = jnp.min(diffs2, axis=-1)
    err_ref[0, 0, 0] = err
    m = (diffs2 == err[:, None]).astype(jnp.bfloat16)
    gA = jnp.dot(m, chiz_ref[...], preferred_element_type=jnp.float32)
    hat = (gA[:, 0:D_K]
           + jnp.dot(m, cmid_ref[...], preferred_element_type=jnp.float32)
           + jnp.dot(m, clo_ref[...], preferred_element_type=jnp.float32))
    z = (4.0 * gA[:, D_K] + gA[:, D_K + 1]).astype(jnp.int32)
    hat_ref[0, 0, 0] = hat
    z_ref[0, 0, 0] = z

    @pl.when(jnp.max(gA[:, D_K + 2]) > 1.5)
    def _():
        iota = jax.lax.broadcasted_iota(jnp.int32, diffs2.shape, 1)
        zt = jnp.min(
            jnp.where(diffs2 == err[:, None], iota, jnp.int32(N_CODE)), axis=-1
        ).astype(jnp.int32)
        onehot = (iota == zt[:, None]).astype(jnp.bfloat16)
        hat_t = (jnp.dot(onehot, chiz_ref[...],
                         preferred_element_type=jnp.float32)[:, 0:D_K]
                 + jnp.dot(onehot, cmid_ref[...],
                           preferred_element_type=jnp.float32)
                 + jnp.dot(onehot, clo_ref[...],
                           preferred_element_type=jnp.float32))
        hat_ref[0, 0, 0] = hat_t
        z_ref[0, 0, 0] = zt


@jax.jit
def kernel(vecs, loss_mask):
    orig_dtype = vecs.dtype
    vecs_hp = vecs.astype(jnp.float32)
    c3 = _codebook()
    c = c3[0]
    ct = c.T
    cnorm = jnp.sum(jnp.square(c3), axis=-1)[0][None, :]
    c_hi = c.astype(jnp.bfloat16)
    r1 = c - jax.lax.optimization_barrier(c_hi).astype(jnp.float32)
    c_mid = r1.astype(jnp.bfloat16)
    c_lo = (r1 - jax.lax.optimization_barrier(c_mid).astype(jnp.float32)
            ).astype(jnp.bfloat16)
    sidx = jnp.arange(N_CODE, dtype=jnp.float32)
    aug = jnp.stack(
        [jnp.floor(sidx / 4.0), jnp.mod(sidx, 4.0), jnp.ones_like(sidx)]
        + [jnp.zeros_like(sidx)] * 5, axis=1)
    chiz = jnp.concatenate([c_hi, aug.astype(jnp.bfloat16)], axis=1)
    vnorm = jnp.sum(jnp.square(vecs_hp), axis=-1)[..., None]

    grid = (T_DIM, B_DIM)
    hat, z, err = pl.pallas_call(
        _vq_tile_kernel,
        grid=grid,
        in_specs=[
            pl.BlockSpec((1, 1, H_DIM, L_DIM, D_K), lambda t, b: (t, b, 0, 0, 0)),
            pl.BlockSpec((1, 1, H_DIM, L_DIM, 1), lambda t, b: (t, b, 0, 0, 0)),
            pl.BlockSpec((D_K, N_CODE), lambda t, b: (0, 0)),
            pl.BlockSpec((N_CODE, D_K + 8), lambda t, b: (0, 0)),
            pl.BlockSpec((N_CODE, D_K), lambda t, b: (0, 0)),
            pl.BlockSpec((N_CODE, D_K), lambda t, b: (0, 0)),
            pl.BlockSpec((1, N_CODE), lambda t, b: (0, 0)),
        ],
        out_specs=[
            pl.BlockSpec((1, 1, H_DIM, L_DIM, D_K), lambda t, b: (t, b, 0, 0, 0)),
            pl.BlockSpec((1, 1, H_DIM, L_DIM), lambda t, b: (t, b, 0, 0)),
            pl.BlockSpec((1, 1, H_DIM, L_DIM), lambda t, b: (t, b, 0, 0)),
        ],
        out_shape=[
            jax.ShapeDtypeStruct((T_DIM, B_DIM, H_DIM, L_DIM, D_K), jnp.float32),
            jax.ShapeDtypeStruct((T_DIM, B_DIM, H_DIM, L_DIM), jnp.int32),
            jax.ShapeDtypeStruct((T_DIM, B_DIM, H_DIM, L_DIM), jnp.float32),
        ],
    )(vecs_hp, vnorm, ct, chiz, c_mid, c_lo, cnorm)

    vecs_hat = hat.astype(orig_dtype)
    errs2 = err
    l_commit = jnp.mean(jnp.sum(loss_mask[:, None, :, :] * errs2, axis=1))
    l_codebook = jnp.zeros((), dtype=jnp.float32)
    return vecs_hat, z, l_commit, l_codebook, errs2


# speedup vs baseline: 3.0558x; 1.1390x over previous
import jax
import jax.numpy as jnp
from jax.experimental import pallas as pl

D_K = 64
N_CODE = 1024
PE_LAM = 100000.0
TAU = float(D_K) ** 0.5

T_DIM, B_DIM, H_DIM, L_DIM = 8, 8, 1, 512


def _codebook():
    pos = jnp.arange(N_CODE, dtype=jnp.float32)
    inv_lams = 1.0 / (PE_LAM ** (jnp.arange(0, D_K, 2, dtype=jnp.float32) / D_K))
    pre = pos[:, None] * inv_lams[None, :]
    cat = jnp.concatenate([jnp.sin(pre), jnp.cos(pre)], axis=-1)
    rms = cat * jax.lax.rsqrt(jnp.mean(jnp.square(cat), axis=-1, keepdims=True) + 1e-6)
    return (TAU ** -0.5) * jax.lax.stop_gradient(rms)[None, ...]


def _vq_tile_kernel(v_ref, vn_ref, ct_ref, chiz_ref, cn_ref,
                    hat_ref, z_ref, err_ref):
    v = v_ref[0, 0, 0]
    dot = jnp.dot(v, ct_ref[...], preferred_element_type=jnp.float32)
    diffs2 = (vn_ref[0, 0, 0] - 2.0 * dot) + cn_ref[...]
    err = jnp.min(diffs2, axis=-1)
    err_ref[0, 0, 0] = err
    m = (diffs2 == err[:, None]).astype(jnp.bfloat16)
    gA = jnp.dot(m, chiz_ref[...], preferred_element_type=jnp.float32)
    hat_ref[0, 0, 0] = gA[:, 0:D_K]
    z_ref[0, 0, 0] = (4.0 * gA[:, D_K] + gA[:, D_K + 1]).astype(jnp.int32)

    @pl.when(jnp.max(gA[:, D_K + 2]) > 1.5)
    def _():
        iota = jax.lax.broadcasted_iota(jnp.int32, diffs2.shape, 1)
        zt = jnp.min(
            jnp.where(diffs2 == err[:, None], iota, jnp.int32(N_CODE)), axis=-1
        ).astype(jnp.int32)
        onehot = (iota == zt[:, None]).astype(jnp.bfloat16)
        hat_t = jnp.dot(onehot, chiz_ref[...],
                        preferred_element_type=jnp.float32)[:, 0:D_K]
        hat_ref[0, 0, 0] = hat_t
        z_ref[0, 0, 0] = zt


@jax.jit
def kernel(vecs, loss_mask):
    orig_dtype = vecs.dtype
    vecs_hp = vecs.astype(jnp.float32)
    c3 = _codebook()
    c = c3[0]
    ct = c.T
    cnorm = jnp.sum(jnp.square(c3), axis=-1)[0][None, :]
    sidx = jnp.arange(N_CODE, dtype=jnp.float32)
    aug = jnp.stack(
        [jnp.floor(sidx / 4.0), jnp.mod(sidx, 4.0), jnp.ones_like(sidx)]
        + [jnp.zeros_like(sidx)] * 5, axis=1)
    chiz = jnp.concatenate(
        [c.astype(jnp.bfloat16), aug.astype(jnp.bfloat16)], axis=1)
    vnorm = jnp.sum(jnp.square(vecs_hp), axis=-1)[..., None]

    grid = (T_DIM, B_DIM)
    hat, z, err = pl.pallas_call(
        _vq_tile_kernel,
        grid=grid,
        in_specs=[
            pl.BlockSpec((1, 1, H_DIM, L_DIM, D_K), lambda t, b: (t, b, 0, 0, 0)),
            pl.BlockSpec((1, 1, H_DIM, L_DIM, 1), lambda t, b: (t, b, 0, 0, 0)),
            pl.BlockSpec((D_K, N_CODE), lambda t, b: (0, 0)),
            pl.BlockSpec((N_CODE, D_K + 8), lambda t, b: (0, 0)),
            pl.BlockSpec((1, N_CODE), lambda t, b: (0, 0)),
        ],
        out_specs=[
            pl.BlockSpec((1, 1, H_DIM, L_DIM, D_K), lambda t, b: (t, b, 0, 0, 0)),
            pl.BlockSpec((1, 1, H_DIM, L_DIM), lambda t, b: (t, b, 0, 0)),
            pl.BlockSpec((1, 1, H_DIM, L_DIM), lambda t, b: (t, b, 0, 0)),
        ],
        out_shape=[
            jax.ShapeDtypeStruct((T_DIM, B_DIM, H_DIM, L_DIM, D_K), jnp.float32),
            jax.ShapeDtypeStruct((T_DIM, B_DIM, H_DIM, L_DIM), jnp.int32),
            jax.ShapeDtypeStruct((T_DIM, B_DIM, H_DIM, L_DIM), jnp.float32),
        ],
    )(vecs_hp, vnorm, ct, chiz, cnorm)

    vecs_hat = hat.astype(orig_dtype)
    errs2 = err
    l_commit = jnp.mean(jnp.sum(loss_mask[:, None, :, :] * errs2, axis=1))
    l_codebook = jnp.zeros((), dtype=jnp.float32)
    return vecs_hat, z, l_commit, l_codebook, errs2


# speedup vs baseline: 3.3259x; 1.0884x over previous
import jax
import jax.numpy as jnp
from jax.experimental import pallas as pl

D_K = 64
N_CODE = 1024
PE_LAM = 100000.0
TAU = float(D_K) ** 0.5

T_DIM, B_DIM, H_DIM, L_DIM = 8, 8, 1, 512


def _codebook():
    pos = jnp.arange(N_CODE, dtype=jnp.float32)
    inv_lams = 1.0 / (PE_LAM ** (jnp.arange(0, D_K, 2, dtype=jnp.float32) / D_K))
    pre = pos[:, None] * inv_lams[None, :]
    cat = jnp.concatenate([jnp.sin(pre), jnp.cos(pre)], axis=-1)
    rms = cat * jax.lax.rsqrt(jnp.mean(jnp.square(cat), axis=-1, keepdims=True) + 1e-6)
    return (TAU ** -0.5) * jax.lax.stop_gradient(rms)[None, ...]


def _vq_tile_kernel(v_ref, vn_ref, ct_ref, chiz_ref, cn_ref,
                    hat_ref, z_ref, err_ref):
    R = 2 * L_DIM
    v = v_ref[0, :, 0].reshape(R, D_K)
    dot = jnp.dot(v, ct_ref[...], preferred_element_type=jnp.float32)
    diffs2 = (vn_ref[0, :, 0].reshape(R, 1) - 2.0 * dot) + cn_ref[...]
    err = jnp.min(diffs2, axis=-1)
    err_ref[0, :, 0] = err.reshape(2, L_DIM)
    m = (diffs2 == err[:, None]).astype(jnp.bfloat16)
    gA = jnp.dot(m, chiz_ref[...], preferred_element_type=jnp.float32)
    hat_ref[0, :, 0] = gA[:, 0:D_K].reshape(2, L_DIM, D_K)
    z_ref[0, :, 0] = (4.0 * gA[:, D_K] + gA[:, D_K + 1]).astype(jnp.int32).reshape(2, L_DIM)

    @pl.when(jnp.max(gA[:, D_K + 2]) > 1.5)
    def _():
        iota = jax.lax.broadcasted_iota(jnp.int32, diffs2.shape, 1)
        zt = jnp.min(
            jnp.where(diffs2 == err[:, None], iota, jnp.int32(N_CODE)), axis=-1
        ).astype(jnp.int32)
        onehot = (iota == zt[:, None]).astype(jnp.bfloat16)
        hat_t = jnp.dot(onehot, chiz_ref[...],
                        preferred_element_type=jnp.float32)[:, 0:D_K]
        hat_ref[0, :, 0] = hat_t.reshape(2, L_DIM, D_K)
        z_ref[0, :, 0] = zt.reshape(2, L_DIM)


@jax.jit
def kernel(vecs, loss_mask):
    orig_dtype = vecs.dtype
    vecs_hp = vecs.astype(jnp.float32)
    c3 = _codebook()
    c = c3[0]
    ct = c.T
    cnorm = jnp.sum(jnp.square(c3), axis=-1)[0][None, :]
    sidx = jnp.arange(N_CODE, dtype=jnp.float32)
    aug = jnp.stack(
        [jnp.floor(sidx / 4.0), jnp.mod(sidx, 4.0), jnp.ones_like(sidx)]
        + [jnp.zeros_like(sidx)] * 5, axis=1)
    chiz = jnp.concatenate(
        [c.astype(jnp.bfloat16), aug.astype(jnp.bfloat16)], axis=1)
    vnorm = jnp.sum(jnp.square(vecs_hp), axis=-1)[..., None]

    grid = (T_DIM, B_DIM // 2)
    hat, z, err = pl.pallas_call(
        _vq_tile_kernel,
        grid=grid,
        in_specs=[
            pl.BlockSpec((1, 2, H_DIM, L_DIM, D_K), lambda t, b: (t, b, 0, 0, 0)),
            pl.BlockSpec((1, 2, H_DIM, L_DIM, 1), lambda t, b: (t, b, 0, 0, 0)),
            pl.BlockSpec((D_K, N_CODE), lambda t, b: (0, 0)),
            pl.BlockSpec((N_CODE, D_K + 8), lambda t, b: (0, 0)),
            pl.BlockSpec((1, N_CODE), lambda t, b: (0, 0)),
        ],
        out_specs=[
            pl.BlockSpec((1, 2, H_DIM, L_DIM, D_K), lambda t, b: (t, b, 0, 0, 0)),
            pl.BlockSpec((1, 2, H_DIM, L_DIM), lambda t, b: (t, b, 0, 0)),
            pl.BlockSpec((1, 2, H_DIM, L_DIM), lambda t, b: (t, b, 0, 0)),
        ],
        out_shape=[
            jax.ShapeDtypeStruct((T_DIM, B_DIM, H_DIM, L_DIM, D_K), jnp.float32),
            jax.ShapeDtypeStruct((T_DIM, B_DIM, H_DIM, L_DIM), jnp.int32),
            jax.ShapeDtypeStruct((T_DIM, B_DIM, H_DIM, L_DIM), jnp.float32),
        ],
    )(vecs_hp, vnorm, ct, chiz, cnorm)

    vecs_hat = hat.astype(orig_dtype)
    errs2 = err
    l_commit = jnp.mean(jnp.sum(loss_mask[:, None, :, :] * errs2, axis=1))
    l_codebook = jnp.zeros((), dtype=jnp.float32)
    return vecs_hat, z, l_commit, l_codebook, errs2
